# baseline (speedup 1.0000x reference)
import functools

import jax
import jax.numpy as jnp
from jax.experimental import pallas as pl
from jax.experimental.pallas import tpu as pltpu


def _gcn_kernel(x_ref, adj_ref, w1_ref, b1_ref, w2_ref, b2_ref,
                out_ref, s1_ref, s2_ref, *, num_i, bm):
    i = pl.program_id(0)

    @pl.when(i == 0)
    def _compute_s1():
        s1_ref[...] = (
            jnp.dot(x_ref[...], w1_ref[...], preferred_element_type=jnp.float32)
            + b1_ref[...]
        )

    @pl.when(i < num_i)
    def _phase1():
        h = jnp.dot(adj_ref[...], s1_ref[...],
                    preferred_element_type=jnp.float32)
        s2_ref[pl.ds(i * bm, bm), :] = (
            jnp.dot(jnp.maximum(h, 0.0), w2_ref[...],
                    preferred_element_type=jnp.float32)
            + b2_ref[...]
        )

    @pl.when(i >= num_i)
    def _phase2():
        o = jnp.dot(adj_ref[...], s2_ref[...],
                    preferred_element_type=jnp.float32)
        m = jnp.max(o, axis=1, keepdims=True)
        lse = jnp.log(jnp.sum(jnp.exp(o - m), axis=1, keepdims=True)) + m
        out_ref[...] = o - lse


@jax.jit
def kernel(x, adj, W1, b1, W2, b2):
    n, f_in = x.shape
    h_dim = W1.shape[1]
    c = W2.shape[1]
    bm = 200 if n % 200 == 0 else 8
    num_i = n // bm

    b1r = b1.reshape(1, h_dim)
    b2r = b2.reshape(1, c)

    return pl.pallas_call(
        functools.partial(_gcn_kernel, num_i=num_i, bm=bm),
        grid=(2 * num_i,),
        in_specs=[
            pl.BlockSpec((n, f_in), lambda i: (0, 0)),
            pl.BlockSpec((bm, n), lambda i, num_i=num_i: (i % num_i, 0)),
            pl.BlockSpec((f_in, h_dim), lambda i: (0, 0)),
            pl.BlockSpec((1, h_dim), lambda i: (0, 0)),
            pl.BlockSpec((h_dim, c), lambda i: (0, 0)),
            pl.BlockSpec((1, c), lambda i: (0, 0)),
        ],
        out_specs=pl.BlockSpec(
            (bm, c), lambda i, num_i=num_i: (jnp.maximum(i - num_i, 0), 0)
        ),
        out_shape=jax.ShapeDtypeStruct((n, c), jnp.float32),
        scratch_shapes=[
            pltpu.VMEM((n, h_dim), jnp.float32),
            pltpu.VMEM((n, c), jnp.float32),
        ],
        compiler_params=pltpu.CompilerParams(
            dimension_semantics=("arbitrary",),
        ),
    )(x, adj, W1, b1r, W2, b2r)
